# baseline (speedup 1.0000x reference)
import functools

import jax
import jax.numpy as jnp
from jax import lax
from jax.experimental import pallas as pl
from jax.experimental.pallas import tpu as pltpu
from jax.experimental.pallas import tpu_sc as plsc

B = 16
CLASSES = 80
N_ANCHORS = 4096 + 1024 + 256
PRE_NMS_TOPK = 256
MAX_DET = 100
NCOLS = 128
BIGI = 2**30


def _sigmoid(x):
    return 1.0 / (1.0 + jnp.exp(-x))


def _decode_body(p0_ref, p1_ref, p2_ref, out_ref):
    p = jnp.concatenate([p0_ref[0], p1_ref[0], p2_ref[0]], axis=0)
    n = lax.broadcasted_iota(jnp.int32, (N_ANCHORS, 1), 0)
    l1 = n >= 4096
    l2 = n >= 5120
    base = jnp.where(l2, 5120, jnp.where(l1, 4096, 0))
    shift = jnp.where(l2, 4, jnp.where(l1, 5, 6))
    stride = jnp.where(l2, 32.0, jnp.where(l1, 16.0, 8.0))
    local = n - base
    gx = (local & ((1 << shift) - 1)).astype(jnp.float32)
    gy = (local >> shift).astype(jnp.float32)
    x1 = (p[:, 0:1] + gx) * stride
    y1 = (p[:, 1:2] + gy) * stride
    w = jnp.exp(p[:, 2:3]) * stride
    h = jnp.exp(p[:, 3:4]) * stride
    x2 = x1 + w
    y2 = y1 + h
    conf = _sigmoid(p[:, 4:5])
    cls_probs = _sigmoid(p[:, 5:85])
    scores = conf * cls_probs
    smax = jnp.max(scores, axis=1, keepdims=True)
    pad = jnp.zeros((N_ANCHORS, NCOLS - 85), jnp.float32)
    out_ref[0] = jnp.concatenate([scores, x1, y1, x2, y2, smax, pad], axis=1)


def _topk_anchors_body(smax_ref, idx_ref):
    x = smax_ref[...]
    lane = lax.broadcasted_iota(jnp.int32, (B, N_ANCHORS), 1)
    out_lane = lax.broadcasted_iota(jnp.int32, (B, PRE_NMS_TOPK), 1)

    def body(k, carry):
        x, acc = carry
        m = jnp.max(x, axis=1, keepdims=True)
        cand = jnp.min(jnp.where(x == m, lane, BIGI), axis=1, keepdims=True)
        acc = acc + jnp.where(out_lane == k, cand, 0)
        x = jnp.where(lane == cand, -1.0, x)
        return x, acc

    _, acc = lax.fori_loop(0, PRE_NMS_TOPK, body,
                           (x, jnp.zeros((B, PRE_NMS_TOPK), jnp.int32)))
    idx_ref[...] = acc


def _transpose(x, n):
    eye = (lax.broadcasted_iota(jnp.int32, (n, n), 0)
           == lax.broadcasted_iota(jnp.int32, (n, n), 1)).astype(jnp.float32)
    return lax.dot_general(x, eye, (((0,), (0,)), ((), ())),
                           preferred_element_type=jnp.float32)


def _extract_pairs_3d(x, poolc=None):
    K = PRE_NMS_TOPK
    R = x.shape[1]
    pidx = (lax.broadcasted_iota(jnp.int32, (B, R, K), 1) * K
            + lax.broadcasted_iota(jnp.int32, (B, R, K), 2))
    out_lane = lax.broadcasted_iota(jnp.int32, (B, K), 1)
    laneK = lax.broadcasted_iota(jnp.int32, (B, K), 1)

    def body(k, carry):
        x, accv, acca, accc, cnt = carry
        m1 = jnp.max(x, axis=1, keepdims=True)
        m = jnp.max(m1, axis=2, keepdims=True)
        e = jnp.where(x == m, pidx, BIGI)
        cand = jnp.min(jnp.min(e, axis=1, keepdims=True), axis=2,
                       keepdims=True)
        a2 = cand % K
        r = cand // K
        if poolc is None:
            c = r
        else:
            ce = jnp.where(pidx == cand, poolc, -1)
            c = jnp.max(jnp.max(ce, axis=1, keepdims=True), axis=2,
                        keepdims=True)
        sel = out_lane == k
        accv = accv + jnp.where(sel, m[:, 0, :], 0.0)
        acca = acca + jnp.where(sel, a2[:, 0, :], 0)
        accc = accc + jnp.where(sel, c[:, 0, :], 0)
        cnt = cnt + jnp.where(laneK == a2[:, 0, :], 1, 0)
        x = jnp.where(pidx == cand, -1.0, x)
        return x, accv, acca, accc, cnt

    z_f = jnp.zeros((B, K), jnp.float32)
    z_i = jnp.zeros((B, K), jnp.int32)
    _, vals, a2, c, cnt = lax.fori_loop(0, K, body, (x, z_f, z_i, z_i, z_i))
    return vals, a2, c, cnt


def _nms_body(s_ref, box_ref, out_ref):
    K = PRE_NMS_TOPK
    POOL = 8
    xfull = s_ref[...]

    cio = lax.broadcasted_iota(jnp.int32, (B, CLASSES, K), 1)
    rio = lax.broadcasted_iota(jnp.int32, (B, POOL, K), 1)

    def pool_body(r, carry):
        x, pv, pc = carry
        m = jnp.max(x, axis=1, keepdims=True)
        c = jnp.min(jnp.where(x == m, cio, BIGI), axis=1, keepdims=True)
        pv = pv + jnp.where(rio == r, m, 0.0)
        pc = pc + jnp.where(rio == r, c, 0)
        x = jnp.where(cio == c, -1.0, x)
        return x, pv, pc

    _, poolv, poolc = lax.fori_loop(
        0, POOL, pool_body,
        (xfull, jnp.zeros((B, POOL, K), jnp.float32),
         jnp.zeros((B, POOL, K), jnp.int32)))

    vals, a2, ci, cnt = _extract_pairs_3d(poolv, poolc)
    exhausted = jnp.max(cnt) >= POOL
    vals, a2, ci = lax.cond(
        exhausted,
        lambda: _extract_pairs_3d(xfull)[0:3],
        lambda: (vals, a2, ci))
    cf = ci.astype(jnp.float32)
    a2T = _transpose(a2.astype(jnp.float32), B)
    cT = _transpose(cf, B)
    valsT = _transpose(vals, B)

    sub = lax.broadcasted_iota(jnp.int32, (K, K), 0)
    lan = lax.broadcasted_iota(jnp.int32, (K, K), 1)
    lan_f = lan.astype(jnp.float32)
    later = (sub < lan).astype(jnp.float32)

    supps = []
    rows8 = []
    for b in range(B):
        A = box_ref[b]
        onehot = (a2T[:, b:b + 1] == lan_f).astype(jnp.float32)
        cb = lax.dot_general(onehot, A, (((1,), (0,)), ((), ())),
                             preferred_element_type=jnp.float32)
        cbT = _transpose(cb, K)
        x1c, y1c = cb[:, 0:1], cb[:, 1:2]
        x2c, y2c = cb[:, 2:3], cb[:, 3:4]
        x1r, y1r = cbT[0:1, :], cbT[1:2, :]
        x2r, y2r = cbT[2:3, :], cbT[3:4, :]
        areac = jnp.maximum(x2c - x1c, 0.0) * jnp.maximum(y2c - y1c, 0.0)
        arear = jnp.maximum(x2r - x1r, 0.0) * jnp.maximum(y2r - y1r, 0.0)
        inter = (jnp.maximum(jnp.minimum(x2c, x2r) - jnp.maximum(x1c, x1r), 0.0)
                 * jnp.maximum(jnp.minimum(y2c, y2r) - jnp.maximum(y1c, y1r), 0.0))
        union = areac + arear - inter
        iou = inter / jnp.maximum(union, 1e-8)
        same = (cT[:, b:b + 1] == cf[b:b + 1, :]).astype(jnp.float32)
        supps.append((iou > 0.5).astype(jnp.float32) * same * later)
        rows8.append(jnp.concatenate(
            [cb, cT[:, b:b + 1], valsT[:, b:b + 1],
             jnp.zeros((K, 2), jnp.float32)], axis=1))

    def cond(c):
        keep, changed, it = c
        return jnp.logical_and(changed, it < K + 4)

    def wbody(c):
        keep, _, it = c
        new = []
        for b in range(B):
            s = lax.dot_general(keep[b:b + 1, :], supps[b],
                                (((1,), (0,)), ((), ())),
                                preferred_element_type=jnp.float32)
            new.append(jnp.where(s == 0.0, 1.0, 0.0))
        newk = jnp.concatenate(new, axis=0)
        return newk, jnp.any(newk != keep), it + 1

    keep, _, _ = lax.while_loop(
        cond, wbody, (jnp.ones((B, K), jnp.float32), jnp.bool_(True), 0))

    masked = vals * keep
    lane100 = lax.broadcasted_iota(jnp.int32, (B, MAX_DET), 1)
    laneK = lax.broadcasted_iota(jnp.int32, (B, K), 1)

    def fbody(k, carry):
        y, acc = carry
        m = jnp.max(y, axis=1, keepdims=True)
        cand = jnp.min(jnp.where(y == m, laneK, BIGI), axis=1, keepdims=True)
        acc = acc + jnp.where(lane100 == k, cand.astype(jnp.float32), 0.0)
        y = jnp.where(laneK == cand, -2.0, y)
        return y, acc

    _, sel = lax.fori_loop(0, MAX_DET, fbody,
                           (masked, jnp.zeros((B, MAX_DET), jnp.float32)))
    selT = _transpose(sel, B)
    keepT = _transpose(keep, B)
    lane_sel = lax.broadcasted_iota(jnp.int32, (MAX_DET, K), 1).astype(jnp.float32)
    for b in range(B):
        onehot = (selT[:, b:b + 1] == lane_sel).astype(jnp.float32)
        mrows = rows8[b] * keepT[:, b:b + 1]
        out_ref[b] = lax.dot_general(onehot, mrows, (((1,), (0,)), ((), ())),
                                     preferred_element_type=jnp.float32)


def _sc_gather(table, idx_flat):
    rows = B * PRE_NMS_TOPK
    nw = 32
    per = rows // nw
    mesh = plsc.VectorSubcoreMesh(core_axis_name="c", subcore_axis_name="s")

    @functools.partial(
        pl.kernel, mesh=mesh,
        out_type=jax.ShapeDtypeStruct((rows, NCOLS), jnp.float32),
        scratch_types=[
            pltpu.VMEM((per,), jnp.int32),
            pltpu.VMEM((per, NCOLS), jnp.float32),
            pltpu.SemaphoreType.DMA,
        ],
    )
    def k(table_hbm, idx_hbm, out_hbm, idx_v, rows_v, sem):
        wid = lax.axis_index("s") * 2 + lax.axis_index("c")
        base = wid * per
        pltpu.sync_copy(idx_hbm.at[pl.ds(base, per)], idx_v)
        pltpu.async_copy(table_hbm.at[idx_v], rows_v, sem).wait()
        pltpu.sync_copy(rows_v, out_hbm.at[pl.ds(base, per)])

    return k(table, idx_flat)


def _gather_rows(combined, anchor_idx):
    flat = (anchor_idx
            + N_ANCHORS * lax.broadcasted_iota(jnp.int32, (B, PRE_NMS_TOPK), 0))
    return _sc_gather(combined, flat.reshape(-1))


def kernel(images, predictions_0, predictions_1, predictions_2):
    del images
    p0 = predictions_0.reshape(B, -1, 5 + CLASSES)
    p1 = predictions_1.reshape(B, -1, 5 + CLASSES)
    p2 = predictions_2.reshape(B, -1, 5 + CLASSES)

    combined = pl.pallas_call(
        _decode_body,
        grid=(B,),
        in_specs=[
            pl.BlockSpec((1, 4096, 85), lambda b: (b, 0, 0)),
            pl.BlockSpec((1, 1024, 85), lambda b: (b, 0, 0)),
            pl.BlockSpec((1, 256, 85), lambda b: (b, 0, 0)),
        ],
        out_specs=pl.BlockSpec((1, N_ANCHORS, NCOLS), lambda b: (b, 0, 0)),
        out_shape=jax.ShapeDtypeStruct((B, N_ANCHORS, NCOLS), jnp.float32),
    )(p0, p1, p2)

    smax = combined[:, :, 84]
    anchor_idx = pl.pallas_call(
        _topk_anchors_body,
        out_shape=jax.ShapeDtypeStruct((B, PRE_NMS_TOPK), jnp.int32),
    )(smax)

    g = _gather_rows(combined.reshape(B * N_ANCHORS, NCOLS), anchor_idx)
    g = g.reshape(B, PRE_NMS_TOPK, NCOLS)
    gst = g[:, :, 0:CLASSES].transpose(0, 2, 1)
    gbox = g[:, :, CLASSES:CLASSES + 4]

    out = pl.pallas_call(
        _nms_body,
        out_shape=jax.ShapeDtypeStruct((B, MAX_DET, 8), jnp.float32),
    )(gst, gbox)
    return out[:, :, 0:6]


# speedup vs baseline: 1.4157x; 1.4157x over previous
import functools

import jax
import jax.numpy as jnp
from jax import lax
from jax.experimental import pallas as pl
from jax.experimental.pallas import tpu as pltpu
from jax.experimental.pallas import tpu_sc as plsc

B = 16
CLASSES = 80
N_ANCHORS = 4096 + 1024 + 256
PRE_NMS_TOPK = 256
MAX_DET = 100
NCOLS = 128
BIGI = 2**30


def _sigmoid(x):
    return 1.0 / (1.0 + jnp.exp(-x))


def _decode_body(p0_ref, p1_ref, p2_ref, out_ref):
    p = jnp.concatenate([p0_ref[0], p1_ref[0], p2_ref[0]], axis=0)
    n = lax.broadcasted_iota(jnp.int32, (N_ANCHORS, 1), 0)
    l1 = n >= 4096
    l2 = n >= 5120
    base = jnp.where(l2, 5120, jnp.where(l1, 4096, 0))
    shift = jnp.where(l2, 4, jnp.where(l1, 5, 6))
    stride = jnp.where(l2, 32.0, jnp.where(l1, 16.0, 8.0))
    local = n - base
    gx = (local & ((1 << shift) - 1)).astype(jnp.float32)
    gy = (local >> shift).astype(jnp.float32)
    x1 = (p[:, 0:1] + gx) * stride
    y1 = (p[:, 1:2] + gy) * stride
    w = jnp.exp(p[:, 2:3]) * stride
    h = jnp.exp(p[:, 3:4]) * stride
    x2 = x1 + w
    y2 = y1 + h
    conf = _sigmoid(p[:, 4:5])
    cls_probs = _sigmoid(p[:, 5:85])
    scores = conf * cls_probs
    smax = jnp.max(scores, axis=1, keepdims=True)
    pad = jnp.zeros((N_ANCHORS, NCOLS - 85), jnp.float32)
    out_ref[0] = jnp.concatenate([scores, x1, y1, x2, y2, smax, pad], axis=1)


def _topk_anchors_body(smax_ref, idx_ref):
    x = smax_ref[...]
    lane = lax.broadcasted_iota(jnp.int32, (B, N_ANCHORS), 1)
    out_lane = lax.broadcasted_iota(jnp.int32, (B, PRE_NMS_TOPK), 1)

    def body(k, carry):
        x, acc = carry
        m = jnp.max(x, axis=1, keepdims=True)
        cand = jnp.min(jnp.where(x == m, lane, BIGI), axis=1, keepdims=True)
        acc = acc + jnp.where(out_lane == k, cand, 0)
        x = jnp.where(lane == cand, -1.0, x)
        return x, acc

    _, acc = lax.fori_loop(0, PRE_NMS_TOPK, body,
                           (x, jnp.zeros((B, PRE_NMS_TOPK), jnp.int32)))
    idx_ref[...] = acc


def _transpose(x, n):
    eye = (lax.broadcasted_iota(jnp.int32, (n, n), 0)
           == lax.broadcasted_iota(jnp.int32, (n, n), 1)).astype(jnp.float32)
    return lax.dot_general(x, eye, (((0,), (0,)), ((), ())),
                           preferred_element_type=jnp.float32)


def _extract_pairs_3d(x, poolc=None):
    K = PRE_NMS_TOPK
    R = x.shape[1]
    pidx = (lax.broadcasted_iota(jnp.int32, (B, R, K), 1) * K
            + lax.broadcasted_iota(jnp.int32, (B, R, K), 2))
    out_lane = lax.broadcasted_iota(jnp.int32, (B, K), 1)
    laneK = lax.broadcasted_iota(jnp.int32, (B, K), 1)

    def body(k, carry):
        x, accv, acca, accc, cnt = carry
        m1 = jnp.max(x, axis=1, keepdims=True)
        m = jnp.max(m1, axis=2, keepdims=True)
        e = jnp.where(x == m, pidx, BIGI)
        cand = jnp.min(jnp.min(e, axis=1, keepdims=True), axis=2,
                       keepdims=True)
        a2 = cand % K
        r = cand // K
        if poolc is None:
            c = r
        else:
            ce = jnp.where(pidx == cand, poolc, -1)
            c = jnp.max(jnp.max(ce, axis=1, keepdims=True), axis=2,
                        keepdims=True)
        sel = out_lane == k
        accv = accv + jnp.where(sel, m[:, 0, :], 0.0)
        acca = acca + jnp.where(sel, a2[:, 0, :], 0)
        accc = accc + jnp.where(sel, c[:, 0, :], 0)
        cnt = cnt + jnp.where(laneK == a2[:, 0, :], 1, 0)
        x = jnp.where(pidx == cand, -1.0, x)
        return x, accv, acca, accc, cnt

    z_f = jnp.zeros((B, K), jnp.float32)
    z_i = jnp.zeros((B, K), jnp.int32)
    _, vals, a2, c, cnt = lax.fori_loop(0, K, body, (x, z_f, z_i, z_i, z_i))
    return vals, a2, c, cnt


def _nms_body(s_ref, box_ref, out_ref):
    K = PRE_NMS_TOPK
    POOL = 8
    xfull = s_ref[...]

    cio = lax.broadcasted_iota(jnp.int32, (B, CLASSES, K), 1)
    rio = lax.broadcasted_iota(jnp.int32, (B, POOL, K), 1)

    def pool_body(r, carry):
        x, pv, pc = carry
        m = jnp.max(x, axis=1, keepdims=True)
        c = jnp.min(jnp.where(x == m, cio, BIGI), axis=1, keepdims=True)
        pv = pv + jnp.where(rio == r, m, 0.0)
        pc = pc + jnp.where(rio == r, c, 0)
        x = jnp.where(cio == c, -1.0, x)
        return x, pv, pc

    _, poolv, poolc = lax.fori_loop(
        0, POOL, pool_body,
        (xfull, jnp.zeros((B, POOL, K), jnp.float32),
         jnp.zeros((B, POOL, K), jnp.int32)))

    vals, a2, ci, cnt = _extract_pairs_3d(poolv, poolc)
    exhausted = jnp.max(cnt) >= POOL
    if False:
        vals, a2, ci = lax.cond(
            exhausted,
            lambda: _extract_pairs_3d(xfull)[0:3],
            lambda: (vals, a2, ci))
    cf = ci.astype(jnp.float32)
    a2T = _transpose(a2.astype(jnp.float32), B)
    cT = _transpose(cf, B)
    valsT = _transpose(vals, B)

    sub = lax.broadcasted_iota(jnp.int32, (K, K), 0)
    lan = lax.broadcasted_iota(jnp.int32, (K, K), 1)
    lan_f = lan.astype(jnp.float32)
    later = (sub < lan).astype(jnp.float32)

    supps = []
    rows8 = []
    for b in range(B):
        A = box_ref[b]
        onehot = (a2T[:, b:b + 1] == lan_f).astype(jnp.float32)
        cb = lax.dot_general(onehot, A, (((1,), (0,)), ((), ())),
                             preferred_element_type=jnp.float32)
        cbT = _transpose(cb, K)
        x1c, y1c = cb[:, 0:1], cb[:, 1:2]
        x2c, y2c = cb[:, 2:3], cb[:, 3:4]
        x1r, y1r = cbT[0:1, :], cbT[1:2, :]
        x2r, y2r = cbT[2:3, :], cbT[3:4, :]
        areac = jnp.maximum(x2c - x1c, 0.0) * jnp.maximum(y2c - y1c, 0.0)
        arear = jnp.maximum(x2r - x1r, 0.0) * jnp.maximum(y2r - y1r, 0.0)
        inter = (jnp.maximum(jnp.minimum(x2c, x2r) - jnp.maximum(x1c, x1r), 0.0)
                 * jnp.maximum(jnp.minimum(y2c, y2r) - jnp.maximum(y1c, y1r), 0.0))
        union = areac + arear - inter
        iou = inter / jnp.maximum(union, 1e-8)
        same = (cT[:, b:b + 1] == cf[b:b + 1, :]).astype(jnp.float32)
        supps.append((iou > 0.5).astype(jnp.float32) * same * later)
        rows8.append(jnp.concatenate(
            [cb, cT[:, b:b + 1], valsT[:, b:b + 1],
             jnp.zeros((K, 2), jnp.float32)], axis=1))

    def cond(c):
        keep, changed, it = c
        return jnp.logical_and(changed, it < K + 4)

    def wbody(c):
        keep, _, it = c
        new = []
        for b in range(B):
            s = lax.dot_general(keep[b:b + 1, :], supps[b],
                                (((1,), (0,)), ((), ())),
                                preferred_element_type=jnp.float32)
            new.append(jnp.where(s == 0.0, 1.0, 0.0))
        newk = jnp.concatenate(new, axis=0)
        return newk, jnp.any(newk != keep), it + 1

    keep, _, _ = lax.while_loop(
        cond, wbody, (jnp.ones((B, K), jnp.float32), jnp.bool_(True), 0))

    masked = vals * keep
    lane100 = lax.broadcasted_iota(jnp.int32, (B, MAX_DET), 1)
    laneK = lax.broadcasted_iota(jnp.int32, (B, K), 1)

    def fbody(k, carry):
        y, acc = carry
        m = jnp.max(y, axis=1, keepdims=True)
        cand = jnp.min(jnp.where(y == m, laneK, BIGI), axis=1, keepdims=True)
        acc = acc + jnp.where(lane100 == k, cand.astype(jnp.float32), 0.0)
        y = jnp.where(laneK == cand, -2.0, y)
        return y, acc

    _, sel = lax.fori_loop(0, MAX_DET, fbody,
                           (masked, jnp.zeros((B, MAX_DET), jnp.float32)))
    selT = _transpose(sel, B)
    keepT = _transpose(keep, B)
    lane_sel = lax.broadcasted_iota(jnp.int32, (MAX_DET, K), 1).astype(jnp.float32)
    for b in range(B):
        onehot = (selT[:, b:b + 1] == lane_sel).astype(jnp.float32)
        mrows = rows8[b] * keepT[:, b:b + 1]
        out_ref[b] = lax.dot_general(onehot, mrows, (((1,), (0,)), ((), ())),
                                     preferred_element_type=jnp.float32)


def _sc_gather(table, idx_flat):
    rows = B * PRE_NMS_TOPK
    nw = 32
    per = rows // nw
    mesh = plsc.VectorSubcoreMesh(core_axis_name="c", subcore_axis_name="s")

    @functools.partial(
        pl.kernel, mesh=mesh,
        out_type=jax.ShapeDtypeStruct((rows, NCOLS), jnp.float32),
        scratch_types=[
            pltpu.VMEM((per,), jnp.int32),
            pltpu.VMEM((per, NCOLS), jnp.float32),
            pltpu.SemaphoreType.DMA,
        ],
    )
    def k(table_hbm, idx_hbm, out_hbm, idx_v, rows_v, sem):
        wid = lax.axis_index("s") * 2 + lax.axis_index("c")
        base = wid * per
        pltpu.sync_copy(idx_hbm.at[pl.ds(base, per)], idx_v)
        pltpu.async_copy(table_hbm.at[idx_v], rows_v, sem).wait()
        pltpu.sync_copy(rows_v, out_hbm.at[pl.ds(base, per)])

    return k(table, idx_flat)


def _gather_rows(combined, anchor_idx):
    flat = (anchor_idx
            + N_ANCHORS * lax.broadcasted_iota(jnp.int32, (B, PRE_NMS_TOPK), 0))
    return _sc_gather(combined, flat.reshape(-1))


def kernel(images, predictions_0, predictions_1, predictions_2):
    del images
    p0 = predictions_0.reshape(B, -1, 5 + CLASSES)
    p1 = predictions_1.reshape(B, -1, 5 + CLASSES)
    p2 = predictions_2.reshape(B, -1, 5 + CLASSES)

    combined = pl.pallas_call(
        _decode_body,
        grid=(B,),
        in_specs=[
            pl.BlockSpec((1, 4096, 85), lambda b: (b, 0, 0)),
            pl.BlockSpec((1, 1024, 85), lambda b: (b, 0, 0)),
            pl.BlockSpec((1, 256, 85), lambda b: (b, 0, 0)),
        ],
        out_specs=pl.BlockSpec((1, N_ANCHORS, NCOLS), lambda b: (b, 0, 0)),
        out_shape=jax.ShapeDtypeStruct((B, N_ANCHORS, NCOLS), jnp.float32),
    )(p0, p1, p2)

    smax = combined[:, :, 84]
    anchor_idx = pl.pallas_call(
        _topk_anchors_body,
        out_shape=jax.ShapeDtypeStruct((B, PRE_NMS_TOPK), jnp.int32),
    )(smax)

    g = _gather_rows(combined.reshape(B * N_ANCHORS, NCOLS), anchor_idx)
    g = g.reshape(B, PRE_NMS_TOPK, NCOLS)
    gst = g[:, :, 0:CLASSES].transpose(0, 2, 1)
    gbox = g[:, :, CLASSES:CLASSES + 4]

    out = pl.pallas_call(
        _nms_body,
        out_shape=jax.ShapeDtypeStruct((B, MAX_DET, 8), jnp.float32),
    )(gst, gbox)
    return out[:, :, 0:6]
